# baseline (speedup 1.0000x reference)
import functools

import jax
import jax.numpy as jnp
from jax import lax
from jax.experimental import pallas as pl
from jax.experimental.pallas import tpu as pltpu
from jax.experimental.pallas import tpu_sc as plsc

_Z = 0.3605741560459137
_T = 0.7618339657783508

_SKEW0, _TAIL0 = -1.8, 1.8
_SKEW1, _TAIL1 = 2.5, 2.5

_mesh = plsc.VectorSubcoreMesh(core_axis_name="c", subcore_axis_name="s")


@functools.partial(
    pl.kernel,
    mesh=_mesh,
    out_type=jax.ShapeDtypeStruct((1, 4), jnp.float32),
    scratch_types=[
        pltpu.VMEM((16,), jnp.float32),
        pltpu.VMEM((16,), jnp.float32),
    ],
)
def _sc_sample(x_hbm, out_hbm, x_v, o_v):
    @pl.when((lax.axis_index("c") == 0) & (lax.axis_index("s") == 0))
    def _():
        pltpu.sync_copy(x_hbm.at[0], x_v.at[pl.ds(0, 4)])
        v = x_v[...]
        p0 = jnp.broadcast_to(v[0], (16,))
        p1 = jnp.broadcast_to(v[1], (16,))
        mu = jnp.broadcast_to(v[2], (16,))
        sigma = jnp.broadcast_to(v[3], (16,))
        cond = p1 > p0 * _T
        choice_f = jnp.where(cond, 1.0, 0.0).astype(jnp.float32)
        skew = jnp.where(cond, _SKEW1, _SKEW0).astype(jnp.float32)
        tail = jnp.where(cond, _TAIL1, _TAIL0).astype(jnp.float32)
        a = (_Z - skew) / tail
        ea = jnp.exp(a)
        sinh = 0.5 * (ea - 1.0 / ea)
        action = mu + sigma * sinh
        lane = lax.iota(jnp.int32, 16)
        out = jnp.where(
            lane == 0, action,
            jnp.where(lane == 1, choice_f, jnp.where(lane == 2, p0, p1)),
        )
        o_v[...] = out
        pltpu.sync_copy(o_v.at[pl.ds(0, 4)], out_hbm.at[0])


def kernel(inputs):
    return _sc_sample(inputs)


# speedup vs baseline: 1.1842x; 1.1842x over previous
import functools
import math

import jax
import jax.numpy as jnp
from jax import lax
from jax.experimental import pallas as pl
from jax.experimental.pallas import tpu as pltpu
from jax.experimental.pallas import tpu_sc as plsc

_Z = 0.3605741560459137
_T = 0.7618339657783508

_S0 = math.sinh((_Z + 1.8) / 1.8)
_S1 = math.sinh((_Z - 2.5) / 2.5)

_mesh = plsc.ScalarSubcoreMesh(axis_name="c", num_cores=1)


@functools.partial(
    pl.kernel,
    mesh=_mesh,
    out_type=jax.ShapeDtypeStruct((1, 4), jnp.float32),
    scratch_types=[
        pltpu.SMEM((4,), jnp.float32),
        pltpu.SMEM((4,), jnp.float32),
    ],
)
def _sc_sample(x_hbm, out_hbm, x_s, o_s):
    pltpu.sync_copy(x_hbm.at[0], x_s)
    p0 = x_s[0]
    p1 = x_s[1]
    mu = x_s[2]
    sigma = x_s[3]
    cond = p1 > p0 * _T
    choice_f = jnp.where(cond, 1.0, 0.0).astype(jnp.float32)
    sinh = jnp.where(cond, _S1, _S0).astype(jnp.float32)
    action = mu + sigma * sinh
    o_s[0] = action
    o_s[1] = choice_f
    o_s[2] = p0
    o_s[3] = p1
    pltpu.sync_copy(o_s, out_hbm.at[0])


def kernel(inputs):
    return _sc_sample(inputs)


# speedup vs baseline: 13.9586x; 11.7872x over previous
import jax
import jax.numpy as jnp
from jax import lax
from jax.experimental import pallas as pl

_Z = 0.3605741560459137
_T = 0.7618339657783508

_SKEW0, _TAIL0 = -1.8, 1.8
_SKEW1, _TAIL1 = 2.5, 2.5


def _body(x_ref, o_ref):
    v = x_ref[...]
    p0 = v[0, 0]
    p1 = v[0, 1]
    mu = v[0, 2]
    sigma = v[0, 3]
    cond = p1 > p0 * _T
    choice_f = jnp.where(cond, 1.0, 0.0).astype(jnp.float32)
    skew = jnp.where(cond, _SKEW1, _SKEW0).astype(jnp.float32)
    tail = jnp.where(cond, _TAIL1, _TAIL0).astype(jnp.float32)
    a = (_Z - skew) / tail
    ea = jnp.exp(a)
    sinh = 0.5 * (ea - 1.0 / ea)
    action = mu + sigma * sinh
    lane = lax.broadcasted_iota(jnp.int32, (1, 4), 1)
    out = jnp.where(
        lane == 0, action,
        jnp.where(lane == 1, choice_f, jnp.where(lane == 2, p0, p1)),
    )
    o_ref[...] = out


def kernel(inputs):
    return pl.pallas_call(
        _body,
        out_shape=jax.ShapeDtypeStruct((1, 4), jnp.float32),
    )(inputs)


# speedup vs baseline: 14.7821x; 1.0590x over previous
import jax
import jax.numpy as jnp
from jax import lax
from jax.experimental import pallas as pl
from jax.experimental.pallas import tpu as pltpu

_Z = 0.3605741560459137
_T = 0.7618339657783508
_SKEW0, _TAIL0 = -1.8, 1.8
_SKEW1, _TAIL1 = 2.5, 2.5


def _body(x_ref, o_ref):
    p0 = x_ref[0, 0]
    p1 = x_ref[0, 1]
    mu = x_ref[0, 2]
    sigma = x_ref[0, 3]
    cond = p1 > p0 * _T
    choice_f = jnp.where(cond, 1.0, 0.0).astype(jnp.float32)
    skew = jnp.where(cond, _SKEW1, _SKEW0).astype(jnp.float32)
    tail = jnp.where(cond, _TAIL1, _TAIL0).astype(jnp.float32)
    a = (_Z - skew) / tail
    ea = jnp.exp(a)
    sinh = 0.5 * (ea - 1.0 / ea)
    action = mu + sigma * sinh
    o_ref[0, 0] = action
    o_ref[0, 1] = choice_f
    o_ref[0, 2] = p0
    o_ref[0, 3] = p1


def kernel(inputs):
    return pl.pallas_call(
        _body,
        in_specs=[pl.BlockSpec(memory_space=pltpu.SMEM)],
        out_specs=pl.BlockSpec(memory_space=pltpu.SMEM),
        out_shape=jax.ShapeDtypeStruct((1, 4), jnp.float32),
    )(inputs)


# speedup vs baseline: 16.4261x; 1.1112x over previous
import math

import jax
import jax.numpy as jnp
from jax import lax
from jax.experimental import pallas as pl
from jax.experimental.pallas import tpu as pltpu

_Z = 0.3605741560459137
_T = 0.7618339657783508
_S0 = math.sinh((_Z + 1.8) / 1.8)
_S1 = math.sinh((_Z - 2.5) / 2.5)


def _body(x_ref, o_ref):
    p0 = x_ref[0, 0]
    p1 = x_ref[0, 1]
    mu = x_ref[0, 2]
    sigma = x_ref[0, 3]
    cond = p1 > p0 * _T
    choice_f = jnp.where(cond, 1.0, 0.0).astype(jnp.float32)
    sinh = jnp.where(cond, _S1, _S0).astype(jnp.float32)
    action = mu + sigma * sinh
    o_ref[0, 0] = action
    o_ref[0, 1] = choice_f
    o_ref[0, 2] = p0
    o_ref[0, 3] = p1


def kernel(inputs):
    return pl.pallas_call(
        _body,
        in_specs=[pl.BlockSpec(memory_space=pltpu.SMEM)],
        out_specs=pl.BlockSpec(memory_space=pltpu.SMEM),
        out_shape=jax.ShapeDtypeStruct((1, 4), jnp.float32),
    )(inputs)


# speedup vs baseline: 16.5080x; 1.0050x over previous
import math

import jax
import jax.numpy as jnp
from jax.experimental import pallas as pl
from jax.experimental.pallas import tpu as pltpu

_Z = 0.3605741560459137
_T = 0.7618339657783508

_S0 = math.sinh((_Z + 1.8) / 1.8)
_S1 = math.sinh((_Z - 2.5) / 2.5)


def _body(x_ref, o_ref):
    p0 = x_ref[0, 0]
    p1 = x_ref[0, 1]
    mu = x_ref[0, 2]
    sigma = x_ref[0, 3]
    cond = p1 > p0 * _T
    choice_f = jnp.where(cond, 1.0, 0.0).astype(jnp.float32)
    sinh = jnp.where(cond, _S1, _S0).astype(jnp.float32)
    action = mu + sigma * sinh
    o_ref[0, 0] = action
    o_ref[0, 1] = choice_f
    o_ref[0, 2] = p0
    o_ref[0, 3] = p1


def kernel(inputs):
    return pl.pallas_call(
        _body,
        in_specs=[pl.BlockSpec(memory_space=pltpu.SMEM)],
        out_specs=pl.BlockSpec(memory_space=pltpu.SMEM),
        out_shape=jax.ShapeDtypeStruct((1, 4), jnp.float32),
    )(inputs)
